# Initial kernel scaffold; baseline (speedup 1.0000x reference)
import jax
import jax.numpy as jnp
from jax.experimental import pallas as pl


def kernel(inputs, params):
    raise NotImplementedError("write your pallas kernel here")


# speedup vs baseline: 4.3929x; 4.3929x over previous
import jax
import jax.numpy as jnp
import numpy as np
from jax import lax
from jax.experimental import pallas as pl
from jax.experimental.pallas import tpu as pltpu
from jax.experimental.pallas import tpu_sc as plsc

_RES = [16, 23, 32, 46, 64, 92, 128, 184, 256, 368, 512, 736]
_NDIM = 3
_NF = 2
_HASH_SIZE = 2 ** 19
_MASK = _HASH_SIZE - 1
_N = 262144
_NLVL = len(_RES)

_P2 = np.int32(np.uint32(2654435761))
_P3 = np.int32(np.uint32(805459861))


def _level_offsets():
    offs, off = [], 0
    for R in _RES:
        p = min(_HASH_SIZE, R ** _NDIM)
        p = int(np.ceil(p / 8) * 8)
        offs.append(off)
        off += p
    offs.append(off)
    return offs

_OFFS = _level_offsets()
_TOTAL_ROWS = _OFFS[-1]
_LEVELS = [(R, _OFFS[i], R ** _NDIM > _HASH_SIZE) for i, R in enumerate(_RES)]

_NC, _NS, _L = 2, 16, 16
_NW = _NC * _NS
_PW = _N // _NW
_C = 128
_NCHUNK = _PW // _C
_NG = _C // _L
_NSTREAM = _NLVL * 8 * _NF


def _body(xt, params, out, xbuf, fracb, idxb, dstb, outb, sem):
    wid = lax.axis_index("c") * _NS + lax.axis_index("s")

    @pl.loop(0, _NCHUNK)
    def _chunk(k):
        base = wid * _PW + k * _C
        for d in range(_NDIM):
            pltpu.sync_copy(xt.at[pl.ds(d * _N + base, _C)], xbuf.at[d])

        @pl.loop(0, _NG)
        def _idx_groups(g):
            sl = pl.ds(g * _L, _L)
            x = xbuf[0, sl]
            y = xbuf[1, sl]
            z = xbuf[2, sl]
            for li, (R, off, is_hash) in enumerate(_LEVELS):
                px = x * jnp.float32(R - 1)
                py = y * jnp.float32(R - 1)
                pz = z * jnp.float32(R - 1)
                ix = px.astype(jnp.int32)
                iy = py.astype(jnp.int32)
                iz = pz.astype(jnp.int32)
                fracb[li, 0, sl] = px - ix.astype(jnp.float32)
                fracb[li, 1, sl] = py - iy.astype(jnp.float32)
                fracb[li, 2, sl] = pz - iz.astype(jnp.float32)
                if is_hash:
                    hy0 = iy * _P2
                    hz0 = iz * _P3
                    xs = (ix, ix + 1)
                    ys = (hy0, hy0 + _P2)
                    zs = (hz0, hz0 + _P3)
                    for c in range(8):
                        h = lax.bitwise_xor(
                            lax.bitwise_xor(xs[c & 1], ys[(c >> 1) & 1]),
                            zs[(c >> 2) & 1])
                        e = ((h & _MASK) << 1) + (2 * off)
                        idxb[(li * 8 + c) * 2, sl] = e
                        idxb[(li * 8 + c) * 2 + 1, sl] = e + 1
                else:
                    yr0 = iy * (2 * R)
                    zr0 = iz * (2 * R * R) + 2 * off
                    xs = (2 * ix, 2 * ix + 2)
                    ys = (yr0, yr0 + 2 * R)
                    zs = (zr0, zr0 + 2 * R * R)
                    for c in range(8):
                        e = xs[c & 1] + ys[(c >> 1) & 1] + zs[(c >> 2) & 1]
                        idxb[(li * 8 + c) * 2, sl] = e
                        idxb[(li * 8 + c) * 2 + 1, sl] = e + 1

        @pl.loop(0, _NSTREAM)
        def _fire(j):
            pltpu.async_copy(params.at[idxb.at[j]], dstb.at[j], sem)

        @pl.loop(0, _NSTREAM)
        def _drain(j):
            pltpu.make_async_copy(params.at[idxb.at[j]], dstb.at[j],
                                  sem).wait()

        @pl.loop(0, _NG)
        def _acc_groups(g):
            sl = pl.ds(g * _L, _L)
            for li in range(_NLVL):
                fx = fracb[li, 0, sl]
                fy = fracb[li, 1, sl]
                fz = fracb[li, 2, sl]
                ax = (1.0 - fx, fx)
                by = (1.0 - fy, fy)
                cz = (1.0 - fz, fz)
                acc0 = acc1 = None
                for c in range(8):
                    w = ax[c & 1] * by[(c >> 1) & 1] * cz[(c >> 2) & 1]
                    g0 = dstb[(li * 8 + c) * 2, sl]
                    g1 = dstb[(li * 8 + c) * 2 + 1, sl]
                    if c == 0:
                        acc0, acc1 = w * g0, w * g1
                    else:
                        acc0, acc1 = acc0 + w * g0, acc1 + w * g1
                outb[2 * li, sl] = acc0
                outb[2 * li + 1, sl] = acc1

        pltpu.sync_copy(outb, out.at[:, pl.ds(base, _C)])


@jax.jit
def kernel(inputs, params):
    xt = inputs.T.reshape(-1)
    pflat = params.reshape(-1)
    run = pl.kernel(
        _body,
        out_type=jax.ShapeDtypeStruct((_NLVL * _NF, _N), jnp.float32),
        mesh=plsc.VectorSubcoreMesh(core_axis_name="c", subcore_axis_name="s"),
        scratch_types=[
            pltpu.VMEM((_NDIM, _C), jnp.float32),
            pltpu.VMEM((_NLVL, _NDIM, _C), jnp.float32),
            pltpu.VMEM((_NSTREAM, _C), jnp.int32),
            pltpu.VMEM((_NSTREAM, _C), jnp.float32),
            pltpu.VMEM((_NLVL * _NF, _C), jnp.float32),
            pltpu.SemaphoreType.DMA,
        ],
    )
    return run(xt, pflat).T
